# baseline (speedup 1.0000x reference)
import functools

import numpy as _np

import jax
import jax.numpy as jnp
from jax import lax
from jax.experimental import pallas as pl
from jax.experimental.pallas import tpu as pltpu
from jax.experimental.pallas import tpu_sc as plsc

F32 = jnp.float32
BF16 = jnp.bfloat16
I32 = jnp.int32

_NC = 2
_NS = 16
_NW = _NC * _NS


def _pack_bf16_words(a, b):
    ua = lax.bitcast_convert_type(a, I32)
    ub = lax.bitcast_convert_type(b, I32)
    ra = ua + 0x7FFF + ((ua >> 16) & 1)
    rb = ub + 0x7FFF + ((ub >> 16) & 1)
    return ((ra >> 16) & 0xFFFF) | (rb & jnp.int32(-65536))


def _build_agg_tables(atom_output, bond_output, a2a_flat, a2b_flat):
    A, H = atom_output.shape
    HW = H // 2
    NB = 16
    G = 4
    AP = 320
    NCH = AP // G
    mesh = plsc.VectorSubcoreMesh(core_axis_name="c", subcore_axis_name="s")

    @functools.partial(
        pl.kernel, mesh=mesh,
        out_type=[jax.ShapeDtypeStruct((A, HW), I32),
                  jax.ShapeDtypeStruct((A, HW), I32)],
        scratch_types=[
            pltpu.VMEM((AP * NB,), I32),
            pltpu.VMEM((AP * NB,), I32),
            pltpu.VMEM((G * NB, H), F32),
            pltpu.VMEM((G * NB, H), F32),
            pltpu.VMEM((G, H), F32),
            pltpu.VMEM((G, HW), I32),
            pltpu.VMEM((G, HW), I32),
            pltpu.SemaphoreType.DMA,
            pltpu.SemaphoreType.DMA,
            pltpu.SemaphoreType.DMA,
            pltpu.SemaphoreType.DMA,
            pltpu.SemaphoreType.DMA,
        ],
    )
    def k(atom_hbm, bond_hbm, a2a_hbm, a2b_hbm, outa_hbm, outb_hbm,
          idx_a, idx_b, nbr_a, nbr_b, selfr, outra, outrb,
          sa, sb, ss, swa, swb):
        wid = lax.axis_index("s") * _NC + lax.axis_index("c")
        base = wid * AP

        pltpu.sync_copy(a2a_hbm.at[pl.ds(base * NB, AP * NB)], idx_a)
        pltpu.sync_copy(a2b_hbm.at[pl.ds(base * NB, AP * NB)], idx_b)

        def valid(c):
            return base + c * G < A

        def gathers_a(c):
            return (
                pltpu.make_async_copy(
                    atom_hbm.at[idx_a.at[pl.ds(c * G * NB, G * NB)]],
                    nbr_a, sa),
                pltpu.make_async_copy(
                    atom_hbm.at[pl.ds(base + c * G, G)], selfr, ss),
            )

        def gathers_b(c):
            return (
                pltpu.make_async_copy(
                    bond_hbm.at[idx_b.at[pl.ds(c * G * NB, G * NB)]],
                    nbr_b, sb),
            )

        def issue(cps):
            for cp in cps:
                cp.start()

        @pl.when(valid(0))
        def _():
            issue(gathers_a(0))
            issue(gathers_b(0))

        def body(c, carry):
            @pl.when(valid(c))
            def _():
                for cp in gathers_a(c):
                    cp.wait()

                @pl.when(c > 0)
                def _():
                    pltpu.make_async_copy(
                        outa_hbm.at[pl.ds(0, G)], outra, swa).wait()

                def col_a(cc, carry2):
                    sa = pl.ds(cc * 32, 16)
                    sb = pl.ds(cc * 32 + 16, 16)
                    so = pl.ds(cc * 16, 16)
                    for a in range(G):
                        acc_a = selfr[a, sa]
                        acc_b = selfr[a, sb]
                        for j in range(NB):
                            acc_a = acc_a + nbr_a[a * NB + j, sa]
                            acc_b = acc_b + nbr_a[a * NB + j, sb]
                        outra[a, so] = _pack_bf16_words(acc_a, acc_b)
                    return carry2
                lax.fori_loop(0, H // 32, col_a, 0)

                @pl.when(valid(c + 1) & (c + 1 < NCH))
                def _():
                    issue(gathers_a(c + 1))
                pltpu.async_copy(
                    outra, outa_hbm.at[pl.ds(base + c * G, G)], swa)

                for cp in gathers_b(c):
                    cp.wait()

                @pl.when(c > 0)
                def _():
                    pltpu.make_async_copy(
                        outb_hbm.at[pl.ds(0, G)], outrb, swb).wait()

                def col_b(cc, carry2):
                    sa = pl.ds(cc * 32, 16)
                    sb = pl.ds(cc * 32 + 16, 16)
                    so = pl.ds(cc * 16, 16)
                    for a in range(G):
                        acc_a = nbr_b[a * NB, sa]
                        acc_b = nbr_b[a * NB, sb]
                        for j in range(1, NB):
                            acc_a = acc_a + nbr_b[a * NB + j, sa]
                            acc_b = acc_b + nbr_b[a * NB + j, sb]
                        outrb[a, so] = _pack_bf16_words(acc_a, acc_b)
                    return carry2
                lax.fori_loop(0, H // 32, col_b, 0)

                @pl.when(valid(c + 1) & (c + 1 < NCH))
                def _():
                    issue(gathers_b(c + 1))
                pltpu.async_copy(
                    outrb, outb_hbm.at[pl.ds(base + c * G, G)], swb)
            return carry

        lax.fori_loop(0, NCH, body, 0)

        @pl.when(valid(0))
        def _():
            pltpu.make_async_copy(outa_hbm.at[pl.ds(0, G)], outra, swa).wait()
            pltpu.make_async_copy(outb_hbm.at[pl.ds(0, G)], outrb, swb).wait()

    return k(atom_output, bond_output, a2a_flat, a2b_flat)


def _build_aggr(agg_atom, agg_bond, atom_output, bond_output, b2a, b2revb):
    A, HW = agg_atom.shape
    E = b2a.shape[0]
    PER_W = E // _NW
    C = 64
    NCH = PER_W // C
    TAIL = PER_W - NCH * C
    EG = 128
    NEG = (PER_W + EG - 1) // EG
    mesh = plsc.VectorSubcoreMesh(core_axis_name="c", subcore_axis_name="s")

    @functools.partial(
        pl.kernel, mesh=mesh,
        out_type=[jax.ShapeDtypeStruct((E, HW), I32),
                  jax.ShapeDtypeStruct((E, HW), I32)],
        scratch_types=[
            pltpu.VMEM((PER_W,), I32),
            pltpu.VMEM((PER_W,), I32),
            pltpu.VMEM((PER_W,), I32),
            pltpu.VMEM((C, HW), I32),
            pltpu.VMEM((C, HW), I32),
            pltpu.VMEM((C, HW), I32),
            pltpu.VMEM((C, HW), I32),
            pltpu.VMEM((C, HW), I32),
            pltpu.VMEM((C, HW), I32),
            pltpu.SemaphoreType.DMA,
            pltpu.SemaphoreType.DMA,
            pltpu.SemaphoreType.DMA,
            pltpu.SemaphoreType.DMA,
            pltpu.SemaphoreType.DMA,
            pltpu.SemaphoreType.DMA,
            pltpu.SemaphoreType.DMA,
        ],
    )
    def k(ga_hbm, gb_hbm, atom_hbm, bond_hbm, b2a_hbm, brev_hbm,
          outa_hbm, outb_hbm,
          ba_all, brev_all, reva_all, x0, y0, x1, y1, z0, z1,
          sx0, sy0, sx1, sy1, sw0, sw1, se):
        wid = lax.axis_index("s") * _NC + lax.axis_index("c")
        base = wid * PER_W

        pltpu.sync_copy(b2a_hbm.at[pl.ds(base, PER_W)], ba_all)
        pltpu.sync_copy(brev_hbm.at[pl.ds(base, PER_W)], brev_all)
        BATCH = 10
        for j0 in range(0, NEG, BATCH):
            js = range(j0, min(j0 + BATCH, NEG))
            cps = []
            for j in js:
                n = min(EG, PER_W - j * EG)
                cps.append(pltpu.async_copy(
                    b2a_hbm.at[brev_all.at[pl.ds(j * EG, n)]],
                    reva_all.at[pl.ds(j * EG, n)], se))
            for cp in cps:
                cp.wait()

        def gathers_a(c):
            return (
                pltpu.make_async_copy(
                    ga_hbm.at[ba_all.at[pl.ds(c * C, C)]], x0, sx0),
                pltpu.make_async_copy(
                    atom_hbm.at[reva_all.at[pl.ds(c * C, C)]], y0, sy0),
            )

        def gathers_b(c):
            return (
                pltpu.make_async_copy(
                    gb_hbm.at[ba_all.at[pl.ds(c * C, C)]], x1, sx1),
                pltpu.make_async_copy(
                    bond_hbm.at[brev_all.at[pl.ds(c * C, C)]], y1, sy1),
            )

        def issue(cps):
            for cp in cps:
                cp.start()

        def drain(hbm, dst, sem):
            pltpu.make_async_copy(hbm.at[pl.ds(0, C)], dst, sem).wait()

        HIMASK = jnp.int32(-65536)

        def word_sub(wx, wy):
            xa = lax.bitcast_convert_type(wx << 16, F32)
            xb = lax.bitcast_convert_type(wx & HIMASK, F32)
            ya = lax.bitcast_convert_type(wy << 16, F32)
            yb = lax.bitcast_convert_type(wy & HIMASK, F32)
            return _pack_bf16_words(xa - ya, xb - yb)

        def subtract(xr, yr, zr):
            def row(r, carry):
                for cc in range(HW // 16):
                    s = pl.ds(cc * 16, 16)
                    zr[r, s] = word_sub(xr[r, s], yr[r, s])
                return carry
            lax.fori_loop(0, C, row, 0)

        issue(gathers_a(0))
        issue(gathers_b(0))

        def body(c, carry):
            ca_x, ca_y = gathers_a(c)
            ca_x.wait()
            ca_y.wait()

            @pl.when(c > 0)
            def _():
                drain(outa_hbm, z0, sw0)
            subtract(x0, y0, z0)

            @pl.when(c + 1 < NCH)
            def _():
                issue(gathers_a(c + 1))
            pltpu.async_copy(z0, outa_hbm.at[pl.ds(base + c * C, C)], sw0)

            cb_x, cb_y = gathers_b(c)
            cb_x.wait()
            cb_y.wait()

            @pl.when(c > 0)
            def _():
                drain(outb_hbm, z1, sw1)
            subtract(x1, y1, z1)

            @pl.when(c + 1 < NCH)
            def _():
                issue(gathers_b(c + 1))
            pltpu.async_copy(z1, outb_hbm.at[pl.ds(base + c * C, C)], sw1)
            return carry

        lax.fori_loop(0, NCH, body, 0)

        drain(outa_hbm, z0, sw0)
        drain(outb_hbm, z1, sw1)

        if TAIL:
            t0 = NCH * C
            pltpu.async_copy(
                ga_hbm.at[ba_all.at[pl.ds(t0, TAIL)]],
                x0.at[pl.ds(0, TAIL)], sx0).wait()
            pltpu.async_copy(
                atom_hbm.at[reva_all.at[pl.ds(t0, TAIL)]],
                y0.at[pl.ds(0, TAIL)], sy0).wait()
            pltpu.async_copy(
                gb_hbm.at[ba_all.at[pl.ds(t0, TAIL)]],
                x1.at[pl.ds(0, TAIL)], sx1).wait()
            pltpu.async_copy(
                bond_hbm.at[brev_all.at[pl.ds(t0, TAIL)]],
                y1.at[pl.ds(0, TAIL)], sy1).wait()

            def trow(r, carry):
                def tcol(cc, carry2):
                    s = pl.ds(cc * 16, 16)
                    z0[r, s] = word_sub(x0[r, s], y0[r, s])
                    z1[r, s] = word_sub(x1[r, s], y1[r, s])
                    return carry2
                return lax.fori_loop(0, HW // 16, tcol, carry)
            lax.fori_loop(0, TAIL, trow, 0)
            pltpu.sync_copy(z0.at[pl.ds(0, TAIL)],
                            outa_hbm.at[pl.ds(base + t0, TAIL)])
            pltpu.sync_copy(z1.at[pl.ds(0, TAIL)],
                            outb_hbm.at[pl.ds(base + t0, TAIL)])

    return k(agg_atom, agg_bond, atom_output, bond_output, b2a, b2revb)


def _ffn_body(f_ref, aga_ref, agb_ref,
              w1ta_ref, w1ba_ref, b1a_ref, w2a_ref, b2a_ref, ga_ref, bea_ref,
              w1tb_ref, w1bb_ref, b1b_ref, w2b_ref, b2b_ref, gb_ref, beb_ref,
              oa_ref, ob_ref):
    f16 = f_ref[...].astype(BF16)
    branches = (
        (aga_ref, w1ta_ref, w1ba_ref, b1a_ref, w2a_ref, b2a_ref,
         ga_ref, bea_ref, oa_ref),
        (agb_ref, w1tb_ref, w1bb_ref, b1b_ref, w2b_ref, b2b_ref,
         gb_ref, beb_ref, ob_ref),
    )
    for agr, w1tr, w1br, b1r, w2r, b2r, gr, ber, outr in branches:
        h = jnp.dot(f16, w1tr[...], preferred_element_type=F32)
        h = h + jnp.dot(agr[...].astype(BF16), w1br[...],
                        preferred_element_type=F32)
        h = jnp.maximum(h + b1r[...], 0.0).astype(BF16)
        y = jnp.dot(h, w2r[...], preferred_element_type=F32) + b2r[...]
        m = jnp.mean(y, axis=1, keepdims=True)
        yc = y - m
        v = jnp.mean(yc * yc, axis=1, keepdims=True)
        outr[...] = yc * lax.rsqrt(v + 1e-5) * gr[...] + ber[...]


def _run_ffn(f_bonds, aggr_a, aggr_b, weights):
    E, H = f_bonds.shape
    D = weights[0].shape[1]
    BE = 1280
    grid = (E // BE,)
    row_spec = pl.BlockSpec((BE, H), lambda i: (i, 0))
    full_spec = lambda r, c: pl.BlockSpec((r, c), lambda i: (0, 0))
    w_specs = [
        full_spec(H, D), full_spec(H, D), full_spec(1, D), full_spec(D, H),
        full_spec(1, H), full_spec(1, H), full_spec(1, H),
        full_spec(H, D), full_spec(H, D), full_spec(1, D), full_spec(D, H),
        full_spec(1, H), full_spec(1, H), full_spec(1, H),
    ]
    return pl.pallas_call(
        _ffn_body,
        grid=grid,
        in_specs=[row_spec, row_spec, row_spec] + w_specs,
        out_specs=[row_spec, row_spec],
        out_shape=[jax.ShapeDtypeStruct((E, H), F32),
                   jax.ShapeDtypeStruct((E, H), F32)],
    )(f_bonds, aggr_a, aggr_b, *weights)


def _readout_body(bia_ref, sizes_ref, feats_ref, w1m_ref, w1f_ref, b1_ref,
                  w2_ref, b2_ref, out_ref, *, n_mols, mol_sz):
    x = bia_ref[...]
    H = x.shape[1]
    xr = x.reshape(n_mols, mol_sz, H)
    sizes = sizes_ref[...]
    offs = lax.broadcasted_iota(I32, (n_mols, mol_sz), 1).astype(F32)
    mask = (offs < sizes).astype(F32)
    mv = (xr * mask[:, :, None]).sum(axis=1) / sizes
    h = jnp.dot(mv, w1m_ref[...], preferred_element_type=F32)
    h = h + jnp.dot(feats_ref[...], w1f_ref[...], preferred_element_type=F32)
    h = jnp.maximum(h + b1_ref[...], 0.0)
    y = jnp.dot(h, w2_ref[...], preferred_element_type=F32) + b2_ref[...]
    out_ref[...] = jax.nn.sigmoid(y) * 0.5


def _run_readout(bond_in_atom_out, sizes_f, features, Wf1, bf1, Wf2, bf2):
    NA, H = bond_in_atom_out.shape
    M, FT = features.shape
    mol_sz = NA // M
    FH = Wf1.shape[1]
    T = Wf2.shape[1]
    body = functools.partial(_readout_body, n_mols=M, mol_sz=mol_sz)
    return pl.pallas_call(
        body,
        out_shape=jax.ShapeDtypeStruct((M, T), F32),
    )(bond_in_atom_out, sizes_f, features,
      Wf1[:H], Wf1[H:], bf1.reshape(1, FH), Wf2, bf2.reshape(1, T))


def kernel(atom_output, bond_output, original_f_atoms, original_f_bonds,
           a2a, a2b, b2a, b2revb, a_scope, features_batch, bond_in_atom_out,
           W1_bfa, b1_bfa, W2_bfa, b2_bfa, g_bfa, be_bfa,
           W1_bfb, b1_bfb, W2_bfb, b2_bfb, g_bfb, be_bfb,
           Wf1, bf1, Wf2, bf2):
    pad_to = _NW * 320 * 16
    a2a_flat = a2a.astype(I32).reshape(-1)
    a2b_flat = a2b.astype(I32).reshape(-1)
    if a2a_flat.shape[0] < pad_to:
        a2a_flat = jnp.pad(a2a_flat, (0, pad_to - a2a_flat.shape[0]))
        a2b_flat = jnp.pad(a2b_flat, (0, pad_to - a2b_flat.shape[0]))
    b2a_i = b2a.astype(I32)
    brev_i = b2revb.astype(I32)

    H = atom_output.shape[1]
    perm = _np.arange(H).reshape(H // 32, 2, 16).transpose(0, 2, 1).reshape(-1)

    def to_words(x):
        xb = x[:, perm].astype(BF16)
        return lax.bitcast_convert_type(xb.reshape(-1, H // 2, 2), I32)

    def from_words(w):
        return lax.bitcast_convert_type(w, BF16).reshape(-1, H)

    atom_w = to_words(atom_output)
    bond_w = to_words(bond_output)

    agg_atom_w, agg_bond_w = _build_agg_tables(
        atom_output, bond_output, a2a_flat, a2b_flat)

    weights = (
        W1_bfa[:H].astype(BF16), W1_bfa[H:][perm].astype(BF16),
        b1_bfa.reshape(1, -1), W2_bfa.astype(BF16),
        b2_bfa.reshape(1, -1), g_bfa.reshape(1, -1), be_bfa.reshape(1, -1),
        W1_bfb[:H].astype(BF16), W1_bfb[H:][perm].astype(BF16),
        b1_bfb.reshape(1, -1), W2_bfb.astype(BF16),
        b2_bfb.reshape(1, -1), g_bfb.reshape(1, -1), be_bfb.reshape(1, -1))

    aggr_aw, aggr_bw = _build_aggr(
        agg_atom_w, agg_bond_w, atom_w, bond_w, b2a_i, brev_i)
    out_a, out_b = _run_ffn(original_f_bonds, from_words(aggr_aw),
                            from_words(aggr_bw), weights)

    sizes_f = a_scope[:, 1].astype(F32).reshape(-1, 1)
    output = _run_readout(bond_in_atom_out, sizes_f, features_batch,
                          Wf1, bf1, Wf2, bf2)
    return output, out_a, out_b


# speedup vs baseline: 1.2460x; 1.2460x over previous
import functools

import numpy as _np

import jax
import jax.numpy as jnp
from jax import lax
from jax.experimental import pallas as pl
from jax.experimental.pallas import tpu as pltpu
from jax.experimental.pallas import tpu_sc as plsc

F32 = jnp.float32
BF16 = jnp.bfloat16
I32 = jnp.int32

_NC = 2
_NS = 16
_NW = _NC * _NS


def _pack_bf16_words(a, b):
    ua = lax.bitcast_convert_type(a, I32)
    ub = lax.bitcast_convert_type(b, I32)
    ra = ua + 0x7FFF + ((ua >> 16) & 1)
    rb = ub + 0x7FFF + ((ub >> 16) & 1)
    return ((ra >> 16) & 0xFFFF) | (rb & jnp.int32(-65536))


def _build_agg_tables(atom_output, bond_output, a2a_flat, a2b_flat):
    A, H = atom_output.shape
    HW = H // 2
    NB = 16
    G = 4
    AP = 320
    NCH = AP // G
    mesh = plsc.VectorSubcoreMesh(core_axis_name="c", subcore_axis_name="s")

    @functools.partial(
        pl.kernel, mesh=mesh,
        out_type=[jax.ShapeDtypeStruct((A, HW), I32),
                  jax.ShapeDtypeStruct((A, HW), I32)],
        scratch_types=[
            pltpu.VMEM((AP * NB,), I32),
            pltpu.VMEM((AP * NB,), I32),
            pltpu.VMEM((G * NB, H), F32),
            pltpu.VMEM((G * NB, H), F32),
            pltpu.VMEM((G, H), F32),
            pltpu.VMEM((G, HW), I32),
            pltpu.VMEM((G, HW), I32),
            pltpu.SemaphoreType.DMA,
            pltpu.SemaphoreType.DMA,
            pltpu.SemaphoreType.DMA,
            pltpu.SemaphoreType.DMA,
            pltpu.SemaphoreType.DMA,
        ],
    )
    def k(atom_hbm, bond_hbm, a2a_hbm, a2b_hbm, outa_hbm, outb_hbm,
          idx_a, idx_b, nbr_a, nbr_b, selfr, outra, outrb,
          sa, sb, ss, swa, swb):
        wid = lax.axis_index("s") * _NC + lax.axis_index("c")
        base = wid * AP

        pltpu.sync_copy(a2a_hbm.at[pl.ds(base * NB, AP * NB)], idx_a)
        pltpu.sync_copy(a2b_hbm.at[pl.ds(base * NB, AP * NB)], idx_b)

        def valid(c):
            return base + c * G < A

        def gathers_a(c):
            return (
                pltpu.make_async_copy(
                    atom_hbm.at[idx_a.at[pl.ds(c * G * NB, G * NB)]],
                    nbr_a, sa),
                pltpu.make_async_copy(
                    atom_hbm.at[pl.ds(base + c * G, G)], selfr, ss),
            )

        def gathers_b(c):
            return (
                pltpu.make_async_copy(
                    bond_hbm.at[idx_b.at[pl.ds(c * G * NB, G * NB)]],
                    nbr_b, sb),
            )

        def issue(cps):
            for cp in cps:
                cp.start()

        @pl.when(valid(0))
        def _():
            issue(gathers_a(0))
            issue(gathers_b(0))

        def body(c, carry):
            @pl.when(valid(c))
            def _():
                for cp in gathers_a(c):
                    cp.wait()

                @pl.when(c > 0)
                def _():
                    pltpu.make_async_copy(
                        outa_hbm.at[pl.ds(0, G)], outra, swa).wait()

                def col_a(cc, carry2):
                    sa = pl.ds(cc * 32, 16)
                    sb = pl.ds(cc * 32 + 16, 16)
                    so = pl.ds(cc * 16, 16)
                    for a in range(G):
                        acc_a = selfr[a, sa]
                        acc_b = selfr[a, sb]
                        for j in range(NB):
                            acc_a = acc_a + nbr_a[a * NB + j, sa]
                            acc_b = acc_b + nbr_a[a * NB + j, sb]
                        outra[a, so] = _pack_bf16_words(acc_a, acc_b)
                    return carry2
                lax.fori_loop(0, H // 32, col_a, 0)

                @pl.when(valid(c + 1) & (c + 1 < NCH))
                def _():
                    issue(gathers_a(c + 1))
                pltpu.async_copy(
                    outra, outa_hbm.at[pl.ds(base + c * G, G)], swa)

                for cp in gathers_b(c):
                    cp.wait()

                @pl.when(c > 0)
                def _():
                    pltpu.make_async_copy(
                        outb_hbm.at[pl.ds(0, G)], outrb, swb).wait()

                def col_b(cc, carry2):
                    sa = pl.ds(cc * 32, 16)
                    sb = pl.ds(cc * 32 + 16, 16)
                    so = pl.ds(cc * 16, 16)
                    for a in range(G):
                        acc_a = nbr_b[a * NB, sa]
                        acc_b = nbr_b[a * NB, sb]
                        for j in range(1, NB):
                            acc_a = acc_a + nbr_b[a * NB + j, sa]
                            acc_b = acc_b + nbr_b[a * NB + j, sb]
                        outrb[a, so] = _pack_bf16_words(acc_a, acc_b)
                    return carry2
                lax.fori_loop(0, H // 32, col_b, 0)

                @pl.when(valid(c + 1) & (c + 1 < NCH))
                def _():
                    issue(gathers_b(c + 1))
                pltpu.async_copy(
                    outrb, outb_hbm.at[pl.ds(base + c * G, G)], swb)
            return carry

        lax.fori_loop(0, NCH, body, 0)

        @pl.when(valid(0))
        def _():
            pltpu.make_async_copy(outa_hbm.at[pl.ds(0, G)], outra, swa).wait()
            pltpu.make_async_copy(outb_hbm.at[pl.ds(0, G)], outrb, swb).wait()

    return k(atom_output, bond_output, a2a_flat, a2b_flat)


def _build_aggr(agg_atom, agg_bond, atom_output, bond_output, b2a, b2revb):
    A, HW = agg_atom.shape
    E = b2a.shape[0]
    PER_W = E // _NW
    C = 64
    NCH = PER_W // C
    TAIL = PER_W - NCH * C
    EG = 128
    NEG = (PER_W + EG - 1) // EG
    mesh = plsc.VectorSubcoreMesh(core_axis_name="c", subcore_axis_name="s")

    @functools.partial(
        pl.kernel, mesh=mesh,
        out_type=[jax.ShapeDtypeStruct((E, HW), I32),
                  jax.ShapeDtypeStruct((E, HW), I32)],
        scratch_types=[
            pltpu.VMEM((PER_W,), I32),
            pltpu.VMEM((PER_W,), I32),
            pltpu.VMEM((PER_W,), I32),
            pltpu.VMEM((C, HW), I32),
            pltpu.VMEM((C, HW), I32),
            pltpu.VMEM((C, HW), I32),
            pltpu.VMEM((C, HW), I32),
            pltpu.VMEM((C, HW), I32),
            pltpu.VMEM((C, HW), I32),
            pltpu.SemaphoreType.DMA,
            pltpu.SemaphoreType.DMA,
            pltpu.SemaphoreType.DMA,
            pltpu.SemaphoreType.DMA,
            pltpu.SemaphoreType.DMA,
            pltpu.SemaphoreType.DMA,
            pltpu.SemaphoreType.DMA,
        ],
    )
    def k(ga_hbm, gb_hbm, atom_hbm, bond_hbm, b2a_hbm, brev_hbm,
          outa_hbm, outb_hbm,
          ba_all, brev_all, reva_all, x0, y0, x1, y1, z0, z1,
          sx0, sy0, sx1, sy1, sw0, sw1, se):
        wid = lax.axis_index("s") * _NC + lax.axis_index("c")
        base = wid * PER_W

        pltpu.sync_copy(b2a_hbm.at[pl.ds(base, PER_W)], ba_all)
        pltpu.sync_copy(brev_hbm.at[pl.ds(base, PER_W)], brev_all)
        BATCH = 10
        for j0 in range(0, NEG, BATCH):
            js = range(j0, min(j0 + BATCH, NEG))
            cps = []
            for j in js:
                n = min(EG, PER_W - j * EG)
                cps.append(pltpu.async_copy(
                    b2a_hbm.at[brev_all.at[pl.ds(j * EG, n)]],
                    reva_all.at[pl.ds(j * EG, n)], se))
            for cp in cps:
                cp.wait()

        def gathers_a(c):
            return (
                pltpu.make_async_copy(
                    ga_hbm.at[ba_all.at[pl.ds(c * C, C)]], x0, sx0),
                pltpu.make_async_copy(
                    atom_hbm.at[reva_all.at[pl.ds(c * C, C)]], y0, sy0),
            )

        def gathers_b(c):
            return (
                pltpu.make_async_copy(
                    gb_hbm.at[ba_all.at[pl.ds(c * C, C)]], x1, sx1),
                pltpu.make_async_copy(
                    bond_hbm.at[brev_all.at[pl.ds(c * C, C)]], y1, sy1),
            )

        def issue(cps):
            for cp in cps:
                cp.start()

        def drain(hbm, dst, sem):
            pltpu.make_async_copy(hbm.at[pl.ds(0, C)], dst, sem).wait()

        HIMASK = jnp.int32(-65536)

        def word_sub(wx, wy):
            xa = lax.bitcast_convert_type(wx << 16, F32)
            xb = lax.bitcast_convert_type(wx & HIMASK, F32)
            ya = lax.bitcast_convert_type(wy << 16, F32)
            yb = lax.bitcast_convert_type(wy & HIMASK, F32)
            return _pack_bf16_words(xa - ya, xb - yb)

        def subtract(xr, yr, zr):
            def row(r, carry):
                for cc in range(HW // 16):
                    s = pl.ds(cc * 16, 16)
                    zr[r, s] = word_sub(xr[r, s], yr[r, s])
                return carry
            lax.fori_loop(0, C, row, 0)

        issue(gathers_a(0))
        issue(gathers_b(0))

        def body(c, carry):
            ca_x, ca_y = gathers_a(c)
            ca_x.wait()
            ca_y.wait()

            @pl.when(c > 0)
            def _():
                drain(outa_hbm, z0, sw0)
            subtract(x0, y0, z0)

            @pl.when(c + 1 < NCH)
            def _():
                issue(gathers_a(c + 1))
            pltpu.async_copy(z0, outa_hbm.at[pl.ds(base + c * C, C)], sw0)

            cb_x, cb_y = gathers_b(c)
            cb_x.wait()
            cb_y.wait()

            @pl.when(c > 0)
            def _():
                drain(outb_hbm, z1, sw1)
            subtract(x1, y1, z1)

            @pl.when(c + 1 < NCH)
            def _():
                issue(gathers_b(c + 1))
            pltpu.async_copy(z1, outb_hbm.at[pl.ds(base + c * C, C)], sw1)
            return carry

        lax.fori_loop(0, NCH, body, 0)

        drain(outa_hbm, z0, sw0)
        drain(outb_hbm, z1, sw1)

        if TAIL:
            t0 = NCH * C
            pltpu.async_copy(
                ga_hbm.at[ba_all.at[pl.ds(t0, TAIL)]],
                x0.at[pl.ds(0, TAIL)], sx0).wait()
            pltpu.async_copy(
                atom_hbm.at[reva_all.at[pl.ds(t0, TAIL)]],
                y0.at[pl.ds(0, TAIL)], sy0).wait()
            pltpu.async_copy(
                gb_hbm.at[ba_all.at[pl.ds(t0, TAIL)]],
                x1.at[pl.ds(0, TAIL)], sx1).wait()
            pltpu.async_copy(
                bond_hbm.at[brev_all.at[pl.ds(t0, TAIL)]],
                y1.at[pl.ds(0, TAIL)], sy1).wait()

            def trow(r, carry):
                def tcol(cc, carry2):
                    s = pl.ds(cc * 16, 16)
                    z0[r, s] = word_sub(x0[r, s], y0[r, s])
                    z1[r, s] = word_sub(x1[r, s], y1[r, s])
                    return carry2
                return lax.fori_loop(0, HW // 16, tcol, carry)
            lax.fori_loop(0, TAIL, trow, 0)
            pltpu.sync_copy(z0.at[pl.ds(0, TAIL)],
                            outa_hbm.at[pl.ds(base + t0, TAIL)])
            pltpu.sync_copy(z1.at[pl.ds(0, TAIL)],
                            outb_hbm.at[pl.ds(base + t0, TAIL)])

    return k(agg_atom, agg_bond, atom_output, bond_output, b2a, b2revb)


def _ffn_body(f_ref, aga_ref, agb_ref,
              w1ta_ref, w1ba_ref, b1a_ref, w2a_ref, b2a_ref, ga_ref, bea_ref,
              w1tb_ref, w1bb_ref, b1b_ref, w2b_ref, b2b_ref, gb_ref, beb_ref,
              oa_ref, ob_ref):
    f16 = f_ref[...].astype(BF16)
    branches = (
        (aga_ref, w1ta_ref, w1ba_ref, b1a_ref, w2a_ref, b2a_ref,
         ga_ref, bea_ref, oa_ref),
        (agb_ref, w1tb_ref, w1bb_ref, b1b_ref, w2b_ref, b2b_ref,
         gb_ref, beb_ref, ob_ref),
    )
    for agr, w1tr, w1br, b1r, w2r, b2r, gr, ber, outr in branches:
        h = jnp.dot(f16, w1tr[...], preferred_element_type=F32)
        h = h + jnp.dot(agr[...].astype(BF16), w1br[...],
                        preferred_element_type=F32)
        h = jnp.maximum(h + b1r[...], 0.0).astype(BF16)
        y = jnp.dot(h, w2r[...], preferred_element_type=F32) + b2r[...]
        m = jnp.mean(y, axis=1, keepdims=True)
        yc = y - m
        v = jnp.mean(yc * yc, axis=1, keepdims=True)
        outr[...] = yc * lax.rsqrt(v + 1e-5) * gr[...] + ber[...]


def _run_ffn(f_bonds, aggr_a, aggr_b, weights):
    E, H = f_bonds.shape
    D = weights[0].shape[1]
    BE = 1280
    grid = (E // BE,)
    row_spec = pl.BlockSpec((BE, H), lambda i: (i, 0))
    full_spec = lambda r, c: pl.BlockSpec((r, c), lambda i: (0, 0))
    w_specs = [
        full_spec(H, D), full_spec(H, D), full_spec(1, D), full_spec(D, H),
        full_spec(1, H), full_spec(1, H), full_spec(1, H),
        full_spec(H, D), full_spec(H, D), full_spec(1, D), full_spec(D, H),
        full_spec(1, H), full_spec(1, H), full_spec(1, H),
    ]
    return pl.pallas_call(
        _ffn_body,
        grid=grid,
        in_specs=[row_spec, row_spec, row_spec] + w_specs,
        out_specs=[row_spec, row_spec],
        out_shape=[jax.ShapeDtypeStruct((E, H), F32),
                   jax.ShapeDtypeStruct((E, H), F32)],
    )(f_bonds, aggr_a, aggr_b, *weights)


def _readout_body(bia_ref, sizes_ref, feats_ref, w1m_ref, w1f_ref, b1_ref,
                  w2_ref, b2_ref, out_ref, *, n_mols, mol_sz):
    x = bia_ref[...]
    H = x.shape[1]
    xr = x.reshape(n_mols, mol_sz, H)
    sizes = sizes_ref[...]
    offs = lax.broadcasted_iota(I32, (n_mols, mol_sz), 1).astype(F32)
    mask = (offs < sizes).astype(F32)
    mv = (xr * mask[:, :, None]).sum(axis=1) / sizes
    h = jnp.dot(mv, w1m_ref[...], preferred_element_type=F32)
    h = h + jnp.dot(feats_ref[...], w1f_ref[...], preferred_element_type=F32)
    h = jnp.maximum(h + b1_ref[...], 0.0)
    y = jnp.dot(h, w2_ref[...], preferred_element_type=F32) + b2_ref[...]
    out_ref[...] = jax.nn.sigmoid(y) * 0.5


def _run_readout(bond_in_atom_out, sizes_f, features, Wf1, bf1, Wf2, bf2):
    NA, H = bond_in_atom_out.shape
    M, FT = features.shape
    mol_sz = NA // M
    FH = Wf1.shape[1]
    T = Wf2.shape[1]
    body = functools.partial(_readout_body, n_mols=M, mol_sz=mol_sz)
    return pl.pallas_call(
        body,
        out_shape=jax.ShapeDtypeStruct((M, T), F32),
    )(bond_in_atom_out, sizes_f, features,
      Wf1[:H], Wf1[H:], bf1.reshape(1, FH), Wf2, bf2.reshape(1, T))


def kernel(atom_output, bond_output, original_f_atoms, original_f_bonds,
           a2a, a2b, b2a, b2revb, a_scope, features_batch, bond_in_atom_out,
           W1_bfa, b1_bfa, W2_bfa, b2_bfa, g_bfa, be_bfa,
           W1_bfb, b1_bfb, W2_bfb, b2_bfb, g_bfb, be_bfb,
           Wf1, bf1, Wf2, bf2):
    pad_to = _NW * 320 * 16
    a2a_flat = a2a.astype(I32).reshape(-1)
    a2b_flat = a2b.astype(I32).reshape(-1)
    if a2a_flat.shape[0] < pad_to:
        a2a_flat = jnp.pad(a2a_flat, (0, pad_to - a2a_flat.shape[0]))
        a2b_flat = jnp.pad(a2b_flat, (0, pad_to - a2b_flat.shape[0]))
    b2a_i = b2a.astype(I32)
    brev_i = b2revb.astype(I32)

    H = atom_output.shape[1]
    perm = _np.arange(H).reshape(H // 32, 2, 16).transpose(0, 2, 1).reshape(-1)

    def to_words(x):
        n = x.shape[0]
        xb = x.reshape(n, H // 32, 2, 16).transpose(0, 1, 3, 2).astype(BF16)
        return lax.bitcast_convert_type(xb, I32).reshape(n, H // 2)

    def from_words(w):
        return lax.bitcast_convert_type(w, BF16).reshape(-1, H)

    atom_w = to_words(atom_output)
    bond_w = to_words(bond_output)

    agg_atom_w, agg_bond_w = _build_agg_tables(
        atom_output, bond_output, a2a_flat, a2b_flat)

    weights = (
        W1_bfa[:H].astype(BF16), W1_bfa[H:][perm].astype(BF16),
        b1_bfa.reshape(1, -1), W2_bfa.astype(BF16),
        b2_bfa.reshape(1, -1), g_bfa.reshape(1, -1), be_bfa.reshape(1, -1),
        W1_bfb[:H].astype(BF16), W1_bfb[H:][perm].astype(BF16),
        b1_bfb.reshape(1, -1), W2_bfb.astype(BF16),
        b2_bfb.reshape(1, -1), g_bfb.reshape(1, -1), be_bfb.reshape(1, -1))

    aggr_aw, aggr_bw = _build_aggr(
        agg_atom_w, agg_bond_w, atom_w, bond_w, b2a_i, brev_i)
    out_a, out_b = _run_ffn(original_f_bonds, from_words(aggr_aw),
                            from_words(aggr_bw), weights)

    sizes_f = a_scope[:, 1].astype(F32).reshape(-1, 1)
    output = _run_readout(bond_in_atom_out, sizes_f, features_batch,
                          Wf1, bf1, Wf2, bf2)
    return output, out_a, out_b


# speedup vs baseline: 2.3114x; 1.8550x over previous
import functools

import jax
import jax.numpy as jnp
from jax import lax
from jax.experimental import pallas as pl
from jax.experimental.pallas import tpu as pltpu
from jax.experimental.pallas import tpu_sc as plsc

F32 = jnp.float32
BF16 = jnp.bfloat16
I32 = jnp.int32

_NC = 2
_NS = 16
_NW = _NC * _NS


def _build_agg_tables(atom_output, bond_output, a2a_flat, a2b_flat):
    A, H = atom_output.shape
    NB = 16
    G = 4
    AP = 320
    NCH = AP // G
    mesh = plsc.VectorSubcoreMesh(core_axis_name="c", subcore_axis_name="s")

    @functools.partial(
        pl.kernel, mesh=mesh,
        out_type=[jax.ShapeDtypeStruct((A, H), F32),
                  jax.ShapeDtypeStruct((A, H), F32)],
        scratch_types=[
            pltpu.VMEM((AP * NB,), I32),
            pltpu.VMEM((AP * NB,), I32),
            pltpu.VMEM((G * NB, H), F32),
            pltpu.VMEM((G * NB, H), F32),
            pltpu.VMEM((G, H), F32),
            pltpu.VMEM((G, H), F32),
            pltpu.VMEM((G, H), F32),
            pltpu.SemaphoreType.DMA,
            pltpu.SemaphoreType.DMA,
            pltpu.SemaphoreType.DMA,
            pltpu.SemaphoreType.DMA,
            pltpu.SemaphoreType.DMA,
        ],
    )
    def k(atom_hbm, bond_hbm, a2a_hbm, a2b_hbm, outa_hbm, outb_hbm,
          idx_a, idx_b, nbr_a, nbr_b, selfr, outra, outrb,
          sa, sb, ss, swa, swb):
        wid = lax.axis_index("s") * _NC + lax.axis_index("c")
        base = wid * AP

        pltpu.sync_copy(a2a_hbm.at[pl.ds(base * NB, AP * NB)], idx_a)
        pltpu.sync_copy(a2b_hbm.at[pl.ds(base * NB, AP * NB)], idx_b)

        def valid(c):
            return base + c * G < A

        def gathers_a(c):
            return (
                pltpu.make_async_copy(
                    atom_hbm.at[idx_a.at[pl.ds(c * G * NB, G * NB)]],
                    nbr_a, sa),
                pltpu.make_async_copy(
                    atom_hbm.at[pl.ds(base + c * G, G)], selfr, ss),
            )

        def gathers_b(c):
            return (
                pltpu.make_async_copy(
                    bond_hbm.at[idx_b.at[pl.ds(c * G * NB, G * NB)]],
                    nbr_b, sb),
            )

        def issue(cps):
            for cp in cps:
                cp.start()

        @pl.when(valid(0))
        def _():
            issue(gathers_a(0))
            issue(gathers_b(0))

        def body(c, carry):
            @pl.when(valid(c))
            def _():
                for cp in gathers_a(c):
                    cp.wait()

                @pl.when(c > 0)
                def _():
                    pltpu.make_async_copy(
                        outa_hbm.at[pl.ds(0, G)], outra, swa).wait()

                def col_a(cc, carry2):
                    s = pl.ds(cc * 16, 16)
                    for a in range(G):
                        acc = selfr[a, s]
                        for j in range(NB):
                            acc = acc + nbr_a[a * NB + j, s]
                        outra[a, s] = acc
                    return carry2
                lax.fori_loop(0, H // 16, col_a, 0)

                @pl.when(valid(c + 1) & (c + 1 < NCH))
                def _():
                    issue(gathers_a(c + 1))
                pltpu.async_copy(
                    outra, outa_hbm.at[pl.ds(base + c * G, G)], swa)

                for cp in gathers_b(c):
                    cp.wait()

                @pl.when(c > 0)
                def _():
                    pltpu.make_async_copy(
                        outb_hbm.at[pl.ds(0, G)], outrb, swb).wait()

                def col_b(cc, carry2):
                    s = pl.ds(cc * 16, 16)
                    for a in range(G):
                        acc = nbr_b[a * NB, s]
                        for j in range(1, NB):
                            acc = acc + nbr_b[a * NB + j, s]
                        outrb[a, s] = acc
                    return carry2
                lax.fori_loop(0, H // 16, col_b, 0)

                @pl.when(valid(c + 1) & (c + 1 < NCH))
                def _():
                    issue(gathers_b(c + 1))
                pltpu.async_copy(
                    outrb, outb_hbm.at[pl.ds(base + c * G, G)], swb)
            return carry

        lax.fori_loop(0, NCH, body, 0)

        @pl.when(valid(0))
        def _():
            pltpu.make_async_copy(outa_hbm.at[pl.ds(0, G)], outra, swa).wait()
            pltpu.make_async_copy(outb_hbm.at[pl.ds(0, G)], outrb, swb).wait()

    return k(atom_output, bond_output, a2a_flat, a2b_flat)


def _build_aggr(agg_atom, agg_bond, atom_output, bond_output, b2a, b2revb):
    A, H = agg_atom.shape
    E = b2a.shape[0]
    PER_W = E // _NW
    C = 32
    NCH = PER_W // C
    TAIL = PER_W - NCH * C
    EG = 128
    NEG = (PER_W + EG - 1) // EG
    mesh = plsc.VectorSubcoreMesh(core_axis_name="c", subcore_axis_name="s")

    @functools.partial(
        pl.kernel, mesh=mesh,
        out_type=[jax.ShapeDtypeStruct((E, H), F32),
                  jax.ShapeDtypeStruct((E, H), F32)],
        scratch_types=[
            pltpu.VMEM((PER_W,), I32),
            pltpu.VMEM((PER_W,), I32),
            pltpu.VMEM((PER_W,), I32),
            pltpu.VMEM((C, H), F32),
            pltpu.VMEM((C, H), F32),
            pltpu.VMEM((C, H), F32),
            pltpu.VMEM((C, H), F32),
            pltpu.VMEM((C, H), F32),
            pltpu.VMEM((C, H), F32),
            pltpu.SemaphoreType.DMA,
            pltpu.SemaphoreType.DMA,
            pltpu.SemaphoreType.DMA,
            pltpu.SemaphoreType.DMA,
            pltpu.SemaphoreType.DMA,
            pltpu.SemaphoreType.DMA,
            pltpu.SemaphoreType.DMA,
        ],
    )
    def k(ga_hbm, gb_hbm, atom_hbm, bond_hbm, b2a_hbm, brev_hbm,
          outa_hbm, outb_hbm,
          ba_all, brev_all, reva_all, x0, y0, x1, y1, z0, z1,
          sx0, sy0, sx1, sy1, sw0, sw1, se):
        wid = lax.axis_index("s") * _NC + lax.axis_index("c")
        base = wid * PER_W

        pltpu.sync_copy(b2a_hbm.at[pl.ds(base, PER_W)], ba_all)
        pltpu.sync_copy(brev_hbm.at[pl.ds(base, PER_W)], brev_all)
        BATCH = 10
        for j0 in range(0, NEG, BATCH):
            js = range(j0, min(j0 + BATCH, NEG))
            cps = []
            for j in js:
                n = min(EG, PER_W - j * EG)
                cps.append(pltpu.async_copy(
                    b2a_hbm.at[brev_all.at[pl.ds(j * EG, n)]],
                    reva_all.at[pl.ds(j * EG, n)], se))
            for cp in cps:
                cp.wait()

        def gathers_a(c):
            return (
                pltpu.make_async_copy(
                    ga_hbm.at[ba_all.at[pl.ds(c * C, C)]], x0, sx0),
                pltpu.make_async_copy(
                    atom_hbm.at[reva_all.at[pl.ds(c * C, C)]], y0, sy0),
            )

        def gathers_b(c):
            return (
                pltpu.make_async_copy(
                    gb_hbm.at[ba_all.at[pl.ds(c * C, C)]], x1, sx1),
                pltpu.make_async_copy(
                    bond_hbm.at[brev_all.at[pl.ds(c * C, C)]], y1, sy1),
            )

        def issue(cps):
            for cp in cps:
                cp.start()

        def drain(hbm, dst, sem):
            pltpu.make_async_copy(hbm.at[pl.ds(0, C)], dst, sem).wait()

        def subtract(xr, yr, zr):
            def row(r, carry):
                for cc in range(H // 16):
                    s = pl.ds(cc * 16, 16)
                    zr[r, s] = xr[r, s] - yr[r, s]
                return carry
            lax.fori_loop(0, C, row, 0)

        issue(gathers_a(0))
        issue(gathers_b(0))

        def body(c, carry):
            ca_x, ca_y = gathers_a(c)
            ca_x.wait()
            ca_y.wait()

            @pl.when(c > 0)
            def _():
                drain(outa_hbm, z0, sw0)
            subtract(x0, y0, z0)

            @pl.when(c + 1 < NCH)
            def _():
                issue(gathers_a(c + 1))
            pltpu.async_copy(z0, outa_hbm.at[pl.ds(base + c * C, C)], sw0)

            cb_x, cb_y = gathers_b(c)
            cb_x.wait()
            cb_y.wait()

            @pl.when(c > 0)
            def _():
                drain(outb_hbm, z1, sw1)
            subtract(x1, y1, z1)

            @pl.when(c + 1 < NCH)
            def _():
                issue(gathers_b(c + 1))
            pltpu.async_copy(z1, outb_hbm.at[pl.ds(base + c * C, C)], sw1)
            return carry

        lax.fori_loop(0, NCH, body, 0)

        drain(outa_hbm, z0, sw0)
        drain(outb_hbm, z1, sw1)

        if TAIL:
            t0 = NCH * C
            pltpu.async_copy(
                ga_hbm.at[ba_all.at[pl.ds(t0, TAIL)]],
                x0.at[pl.ds(0, TAIL)], sx0).wait()
            pltpu.async_copy(
                atom_hbm.at[reva_all.at[pl.ds(t0, TAIL)]],
                y0.at[pl.ds(0, TAIL)], sy0).wait()
            pltpu.async_copy(
                gb_hbm.at[ba_all.at[pl.ds(t0, TAIL)]],
                x1.at[pl.ds(0, TAIL)], sx1).wait()
            pltpu.async_copy(
                bond_hbm.at[brev_all.at[pl.ds(t0, TAIL)]],
                y1.at[pl.ds(0, TAIL)], sy1).wait()

            def trow(r, carry):
                def tcol(cc, carry2):
                    s = pl.ds(cc * 16, 16)
                    z0[r, s] = x0[r, s] - y0[r, s]
                    z1[r, s] = x1[r, s] - y1[r, s]
                    return carry2
                return lax.fori_loop(0, H // 16, tcol, carry)
            lax.fori_loop(0, TAIL, trow, 0)
            pltpu.sync_copy(z0.at[pl.ds(0, TAIL)],
                            outa_hbm.at[pl.ds(base + t0, TAIL)])
            pltpu.sync_copy(z1.at[pl.ds(0, TAIL)],
                            outb_hbm.at[pl.ds(base + t0, TAIL)])

    return k(agg_atom, agg_bond, atom_output, bond_output, b2a, b2revb)


def _ffn_body(f_ref, aga_ref, agb_ref,
              w1ta_ref, w1ba_ref, b1a_ref, w2a_ref, b2a_ref, ga_ref, bea_ref,
              w1tb_ref, w1bb_ref, b1b_ref, w2b_ref, b2b_ref, gb_ref, beb_ref,
              oa_ref, ob_ref):
    f16 = f_ref[...].astype(BF16)
    branches = (
        (aga_ref, w1ta_ref, w1ba_ref, b1a_ref, w2a_ref, b2a_ref,
         ga_ref, bea_ref, oa_ref),
        (agb_ref, w1tb_ref, w1bb_ref, b1b_ref, w2b_ref, b2b_ref,
         gb_ref, beb_ref, ob_ref),
    )
    for agr, w1tr, w1br, b1r, w2r, b2r, gr, ber, outr in branches:
        h = jnp.dot(f16, w1tr[...], preferred_element_type=F32)
        h = h + jnp.dot(agr[...].astype(BF16), w1br[...],
                        preferred_element_type=F32)
        h = jnp.maximum(h + b1r[...], 0.0).astype(BF16)
        y = jnp.dot(h, w2r[...], preferred_element_type=F32) + b2r[...]
        m = jnp.mean(y, axis=1, keepdims=True)
        yc = y - m
        v = jnp.mean(yc * yc, axis=1, keepdims=True)
        outr[...] = yc * lax.rsqrt(v + 1e-5) * gr[...] + ber[...]


def _run_ffn(f_bonds, aggr_a, aggr_b, weights):
    E, H = f_bonds.shape
    D = weights[0].shape[1]
    BE = 1280
    grid = (E // BE,)
    row_spec = pl.BlockSpec((BE, H), lambda i: (i, 0))
    full_spec = lambda r, c: pl.BlockSpec((r, c), lambda i: (0, 0))
    w_specs = [
        full_spec(H, D), full_spec(H, D), full_spec(1, D), full_spec(D, H),
        full_spec(1, H), full_spec(1, H), full_spec(1, H),
        full_spec(H, D), full_spec(H, D), full_spec(1, D), full_spec(D, H),
        full_spec(1, H), full_spec(1, H), full_spec(1, H),
    ]
    return pl.pallas_call(
        _ffn_body,
        grid=grid,
        in_specs=[row_spec, row_spec, row_spec] + w_specs,
        out_specs=[row_spec, row_spec],
        out_shape=[jax.ShapeDtypeStruct((E, H), F32),
                   jax.ShapeDtypeStruct((E, H), F32)],
    )(f_bonds, aggr_a, aggr_b, *weights)


def _readout_body(bia_ref, sizes_ref, feats_ref, w1m_ref, w1f_ref, b1_ref,
                  w2_ref, b2_ref, out_ref, *, n_mols, mol_sz):
    x = bia_ref[...]
    H = x.shape[1]
    xr = x.reshape(n_mols, mol_sz, H)
    sizes = sizes_ref[...]
    offs = lax.broadcasted_iota(I32, (n_mols, mol_sz), 1).astype(F32)
    mask = (offs < sizes).astype(F32)
    mv = (xr * mask[:, :, None]).sum(axis=1) / sizes
    h = jnp.dot(mv, w1m_ref[...], preferred_element_type=F32)
    h = h + jnp.dot(feats_ref[...], w1f_ref[...], preferred_element_type=F32)
    h = jnp.maximum(h + b1_ref[...], 0.0)
    y = jnp.dot(h, w2_ref[...], preferred_element_type=F32) + b2_ref[...]
    out_ref[...] = jax.nn.sigmoid(y) * 0.5


def _run_readout(bond_in_atom_out, sizes_f, features, Wf1, bf1, Wf2, bf2):
    NA, H = bond_in_atom_out.shape
    M, FT = features.shape
    mol_sz = NA // M
    FH = Wf1.shape[1]
    T = Wf2.shape[1]
    body = functools.partial(_readout_body, n_mols=M, mol_sz=mol_sz)
    return pl.pallas_call(
        body,
        out_shape=jax.ShapeDtypeStruct((M, T), F32),
    )(bond_in_atom_out, sizes_f, features,
      Wf1[:H], Wf1[H:], bf1.reshape(1, FH), Wf2, bf2.reshape(1, T))


def kernel(atom_output, bond_output, original_f_atoms, original_f_bonds,
           a2a, a2b, b2a, b2revb, a_scope, features_batch, bond_in_atom_out,
           W1_bfa, b1_bfa, W2_bfa, b2_bfa, g_bfa, be_bfa,
           W1_bfb, b1_bfb, W2_bfb, b2_bfb, g_bfb, be_bfb,
           Wf1, bf1, Wf2, bf2):
    pad_to = _NW * 320 * 16
    a2a_flat = a2a.astype(I32).reshape(-1)
    a2b_flat = a2b.astype(I32).reshape(-1)
    if a2a_flat.shape[0] < pad_to:
        a2a_flat = jnp.pad(a2a_flat, (0, pad_to - a2a_flat.shape[0]))
        a2b_flat = jnp.pad(a2b_flat, (0, pad_to - a2b_flat.shape[0]))
    b2a_i = b2a.astype(I32)
    brev_i = b2revb.astype(I32)

    H = atom_output.shape[1]
    agg_atom, agg_bond = _build_agg_tables(
        atom_output, bond_output, a2a_flat, a2b_flat)

    weights = (
        W1_bfa[:H].astype(BF16), W1_bfa[H:].astype(BF16),
        b1_bfa.reshape(1, -1), W2_bfa.astype(BF16),
        b2_bfa.reshape(1, -1), g_bfa.reshape(1, -1), be_bfa.reshape(1, -1),
        W1_bfb[:H].astype(BF16), W1_bfb[H:].astype(BF16),
        b1_bfb.reshape(1, -1), W2_bfb.astype(BF16),
        b2_bfb.reshape(1, -1), g_bfb.reshape(1, -1), be_bfb.reshape(1, -1))

    aggr_a, aggr_b = _build_aggr(
        agg_atom, agg_bond, atom_output, bond_output, b2a_i, brev_i)
    out_a, out_b = _run_ffn(original_f_bonds, aggr_a, aggr_b, weights)

    sizes_f = a_scope[:, 1].astype(F32).reshape(-1, 1)
    output = _run_readout(bond_in_atom_out, sizes_f, features_batch,
                          Wf1, bf1, Wf2, bf2)
    return output, out_a, out_b
